# baseline (speedup 1.0000x reference)
import functools

import jax
import jax.numpy as jnp
from jax import lax
from jax.experimental import pallas as pl
from jax.experimental.pallas import tpu as pltpu
from jax.experimental.pallas import tpu_sc as plsc

VOCAB = 1000000
EMB = 32
C = 6
B = 16384
NC = 2
NS = 16
NW = NC * NS
BPW = B // NW
CPW = BPW * C
CHUNK = 128

_mesh = plsc.VectorSubcoreMesh(core_axis_name="c", subcore_axis_name="s")


@functools.partial(
    pl.kernel,
    mesh=_mesh,
    compiler_params=pltpu.CompilerParams(
        needs_layout_passes=False, use_tc_tiling_on_sc=False),
    out_type=jax.ShapeDtypeStruct((B * C,), jnp.float32),
    scratch_types=[
        pltpu.VMEM((BPW,), jnp.int32),
        pltpu.VMEM((CPW,), jnp.int32),
        pltpu.VMEM((BPW, EMB), jnp.float32),
        pltpu.VMEM((CPW, EMB), jnp.float32),
        pltpu.VMEM((CPW,), jnp.float32),
        pltpu.SemaphoreType.DMA,
    ],
)
def _w2v(tgt_hbm, ctx_hbm, ttab_hbm, ctab_hbm, out_hbm,
         tidx_v, cidx_v, trow_v, crow_v, out_v, sem):
    wid = lax.axis_index("s") * NC + lax.axis_index("c")
    tbase = wid * BPW
    cbase = wid * CPW

    pltpu.sync_copy(tgt_hbm.at[pl.ds(tbase, BPW)], tidx_v)
    pltpu.sync_copy(ctx_hbm.at[pl.ds(cbase, CPW)], cidx_v)

    handles = []
    for j in range(BPW // CHUNK):
        handles.append(pltpu.async_copy(
            ttab_hbm.at[tidx_v.at[pl.ds(j * CHUNK, CHUNK)]],
            trow_v.at[pl.ds(j * CHUNK, CHUNK)], sem))
    for j in range(CPW // CHUNK):
        handles.append(pltpu.async_copy(
            ctab_hbm.at[cidx_v.at[pl.ds(j * CHUNK, CHUNK)]],
            crow_v.at[pl.ds(j * CHUNK, CHUNK)], sem))
    for h in handles:
        h.wait()

    iota16 = lax.iota(jnp.int32, 16)

    def body(blk, carry):
        b_vec = blk * 16 + iota16
        r_vecs = [b_vec * C + c for c in range(C)]
        acc = [jnp.zeros((16,), jnp.float32) for _ in range(C)]
        for e in range(EMB):
            col = jnp.full((16,), e, jnp.int32)
            tv = plsc.load_gather(trow_v, [b_vec, col])
            for c in range(C):
                cv = plsc.load_gather(crow_v, [r_vecs[c], col])
                acc[c] = acc[c] + tv * cv
        for c in range(C):
            plsc.store_scatter(out_v, [r_vecs[c]], acc[c])
        return carry

    lax.fori_loop(0, BPW // 16, body, 0)

    pltpu.sync_copy(out_v, out_hbm.at[pl.ds(cbase, CPW)])


def kernel(tgt, ctx, target_table, context_table):
    out = _w2v(tgt.reshape(-1), ctx.reshape(-1), target_table, context_table)
    return out.reshape(B, C)
